# baseline (speedup 1.0000x reference)
import numpy as np

import jax
import jax.numpy as jnp
from jax.experimental import pallas as pl
from jax.experimental.pallas import tpu as pltpu

_N_TERMS = 64
_N_OUT = 4
_MASKS = [np.ones(_N_TERMS, dtype=bool) for _ in range(_N_OUT)]

_CHUNKS = 8


def _gather_cols_kernel(x_hbm, o_hbm, sems):
    rows = x_hbm.shape[0] // _CHUNKS
    for i in range(_CHUNKS):
        pltpu.make_async_copy(
            x_hbm.at[pl.ds(i * rows, rows), :],
            o_hbm.at[pl.ds(i * rows, rows), :],
            sems.at[i],
        ).start()
    for i in range(_CHUNKS):
        pltpu.make_async_copy(
            x_hbm.at[pl.ds(i * rows, rows), :],
            o_hbm.at[pl.ds(i * rows, rows), :],
            sems.at[i],
        ).wait()


def _masked_gather(thetas, cols):
    n, _ = thetas.shape
    w = int(cols.shape[0])
    return pl.pallas_call(
        _gather_cols_kernel,
        in_specs=[pl.BlockSpec(memory_space=pl.ANY)],
        out_specs=pl.BlockSpec(memory_space=pl.ANY),
        out_shape=jax.ShapeDtypeStruct((n, w), thetas.dtype),
        scratch_shapes=[pltpu.SemaphoreType.DMA((_CHUNKS,))],
    )(thetas)


def kernel(thetas, time_derivs, coeff_0, coeff_1, coeff_2, coeff_3):
    cols = np.nonzero(_MASKS[0])[0].astype(np.int32)
    gathered = _masked_gather(thetas, cols)
    sparse_thetas = (gathered,) * _N_OUT
    return sparse_thetas + (coeff_0, coeff_1, coeff_2, coeff_3)
